# Initial kernel scaffold; baseline (speedup 1.0000x reference)
import jax
import jax.numpy as jnp
from jax.experimental import pallas as pl


def kernel(x, edge_index, edge_type, edge_norm, weight, w_comp):
    raise NotImplementedError("write your pallas kernel here")


# speedup vs baseline: 1.7684x; 1.7684x over previous
import jax
import jax.numpy as jnp
from jax import lax
from jax.experimental import pallas as pl
from jax.experimental.pallas import tpu as pltpu
from jax.experimental.pallas import tpu_sc as plsc

N_NODES = 10000
N_EDGES = 160000
IN_FEAT = 256
OUT_FEAT = 256
NUM_RELS = 16
NUM_BASES = 8

HALF = 128
NC = 2
NS = 16
BATCH = 128
NBATCH = 79
EPT = NBATCH * BATCH
EPAD = EPT * NS
APAD = 10112
RPT = APAD // NS
ZCHUNKS = (128, 128, 128, 128, 120)


def _wfull_body(c_ref, w_ref, o_ref):
    o_ref[...] = jnp.dot(c_ref[...], w_ref[...],
                         preferred_element_type=jnp.float32)


def _proj_body(x_ref, wf_ref, o_ref):
    o_ref[...] = jnp.dot(x_ref[...], wf_ref[0],
                         preferred_element_type=jnp.float32)


def _sc_body(proj, src_h, type_h, norm_h, dst_h, out,
             acc, sb, tb, nb, db, gb, rows, sem):
    c = lax.axis_index("c")
    s = lax.axis_index("s")
    zf = jnp.zeros((16,), jnp.float32)

    def zrow(jj, carry):
        for k in range(HALF // 16):
            rows[jj, pl.ds(k * 16, 16)] = zf
        return carry
    lax.fori_loop(0, BATCH, zrow, 0)

    abase = s * RPT
    off = 0
    for zc in ZCHUNKS:
        pltpu.sync_copy(rows.at[pl.ds(0, zc)],
                        acc.at[pl.ds(abase + off, zc)])
        off += zc

    plsc.subcore_barrier()

    crn = c * (NUM_RELS * N_NODES)
    ebase = s * EPT

    def mbody(jb, carry):
        eb = ebase + jb * BATCH
        pltpu.sync_copy(src_h.at[pl.ds(eb, BATCH)], sb)
        pltpu.sync_copy(type_h.at[pl.ds(eb, BATCH)], tb)
        pltpu.sync_copy(norm_h.at[pl.ds(eb, BATCH)], nb)
        pltpu.sync_copy(dst_h.at[pl.ds(eb, BATCH)], db.at[0])
        for t in range(BATCH // 16):
            gb[0, pl.ds(t * 16, 16)] = (
                tb[pl.ds(t * 16, 16)] * N_NODES + sb[pl.ds(t * 16, 16)] + crn)

        pltpu.async_copy(proj.at[gb.at[0]], rows, sem).wait()

        def scale(gg, c2):
            nv16 = nb[pl.ds(gg * 16, 16)]
            for k in range(16):
                nvk = jnp.full((16,), nv16[k], jnp.float32)
                row = gg * 16 + k
                for h in range(HALF // 16):
                    rows[row, pl.ds(h * 16, 16)] = (
                        rows[row, pl.ds(h * 16, 16)] * nvk)
            return c2
        lax.fori_loop(0, BATCH // 16, scale, 0)

        pltpu.sync_copy(rows, acc.at[db.at[0]], add=True)
        return carry
    lax.fori_loop(0, NBATCH, mbody, 0)

    plsc.subcore_barrier()

    obase = c * APAD + s * RPT
    off = 0
    for zc in ZCHUNKS:
        pltpu.sync_copy(acc.at[pl.ds(abase + off, zc)],
                        out.at[pl.ds(obase + off, zc)])
        off += zc


_sc_scatter_cache = []


def _get_sc_scatter():
    if not _sc_scatter_cache:
        _sc_scatter_cache.append(pl.kernel(
            _sc_body,
            out_type=jax.ShapeDtypeStruct((NC * APAD, HALF), jnp.float32),
            mesh=plsc.VectorSubcoreMesh(core_axis_name="c",
                                        subcore_axis_name="s",
                                        num_cores=NC, num_subcores=NS),
            scratch_types=[
                pltpu.VMEM_SHARED((APAD, HALF), jnp.float32),
                pltpu.VMEM((BATCH,), jnp.int32),
                pltpu.VMEM((BATCH,), jnp.int32),
                pltpu.VMEM((BATCH,), jnp.float32),
                pltpu.VMEM((1, BATCH), jnp.int32),
                pltpu.VMEM((1, BATCH), jnp.int32),
                pltpu.VMEM((BATCH, HALF), jnp.float32),
                pltpu.SemaphoreType.DMA,
            ],
        ))
    return _sc_scatter_cache[0]


@jax.jit
def kernel(x, edge_index, edge_type, edge_norm, weight, w_comp):
    wfull_flat = pl.pallas_call(
        _wfull_body,
        out_shape=jax.ShapeDtypeStruct((NUM_RELS, IN_FEAT * OUT_FEAT),
                                       jnp.float32),
    )(w_comp, weight.reshape(NUM_BASES, IN_FEAT * OUT_FEAT))
    wfull = wfull_flat.reshape(NUM_RELS, IN_FEAT, OUT_FEAT)

    bn = 2000
    nb = N_NODES // bn
    proj = pl.pallas_call(
        _proj_body,
        grid=(nb, NUM_RELS, NC),
        in_specs=[
            pl.BlockSpec((bn, IN_FEAT), lambda i, r, h: (i, 0)),
            pl.BlockSpec((1, IN_FEAT, HALF), lambda i, r, h: (r, 0, h)),
        ],
        out_specs=pl.BlockSpec(
            (bn, HALF), lambda i, r, h: ((h * NUM_RELS + r) * nb + i, 0)),
        out_shape=jax.ShapeDtypeStruct((NC * NUM_RELS * N_NODES, HALF),
                                       jnp.float32),
    )(x, wfull)

    pad = EPAD - N_EDGES
    src = jnp.pad(edge_index[0], (0, pad))
    dst = jnp.pad(edge_index[1], (0, pad))
    etype = jnp.pad(edge_type, (0, pad))
    norm = jnp.pad(edge_norm.reshape(N_EDGES), (0, pad))
    out2 = _get_sc_scatter()(proj, src, etype, norm, dst)
    return (out2.reshape(NC, APAD, HALF)[:, :N_NODES]
            .transpose(1, 0, 2)
            .reshape(N_NODES, OUT_FEAT))


# speedup vs baseline: 2.4539x; 1.3876x over previous
import jax
import jax.numpy as jnp
from jax import lax
from jax.experimental import pallas as pl
from jax.experimental.pallas import tpu as pltpu
from jax.experimental.pallas import tpu_sc as plsc

N_NODES = 10000
N_EDGES = 160000
IN_FEAT = 256
OUT_FEAT = 256
NUM_RELS = 16
NUM_BASES = 8

HALF = 128
NC = 2
NS = 16
BATCH = 128
NBATCH = 79
EPT = NBATCH * BATCH
EPAD = EPT * NS
APAD = 10112
RPT = APAD // NS
ZCHUNKS = (128, 128, 128, 128, 120)


def _wfull_body(c_ref, w_ref, o_ref):
    o_ref[...] = jnp.dot(c_ref[...], w_ref[...],
                         preferred_element_type=jnp.float32)


def _proj_body(x_ref, wf_ref, o_ref):
    o_ref[...] = jnp.dot(x_ref[...], wf_ref[0],
                         preferred_element_type=jnp.float32)


def _sc_body(proj, meta_h, norm_h, out,
             acc, mb, nb, gb, rows,
             msem0, msem1, nsem0, nsem1, gsem0, gsem1):
    c = lax.axis_index("c")
    s = lax.axis_index("s")
    zf = jnp.zeros((16,), jnp.float32)
    msem = (msem0, msem1)
    nsem = (nsem0, nsem1)
    gsem = (gsem0, gsem1)

    def zrow(jj, carry):
        for k in range(HALF // 16):
            rows[0, jj, pl.ds(k * 16, 16)] = zf
        return carry
    lax.fori_loop(0, BATCH, zrow, 0)

    abase = s * RPT
    off = 0
    for zc in ZCHUNKS:
        pltpu.sync_copy(rows.at[0, pl.ds(0, zc)],
                        acc.at[pl.ds(abase + off, zc)])
        off += zc

    plsc.subcore_barrier()

    crn = c * (NUM_RELS * N_NODES)
    jbase = s * NBATCH

    def build_idx(b, j):
        del j
        for t in range(BATCH // 16):
            sl = pl.ds(t * 16, 16)
            gb[b, sl] = mb[b, 1, sl] * N_NODES + mb[b, 0, sl] + crn

    def start_meta(b, j):
        pltpu.async_copy(meta_h.at[jbase + j], mb.at[b], msem[b])
        pltpu.async_copy(norm_h.at[jbase + j], nb.at[b], nsem[b])

    def wait_meta(b, j):
        pltpu.make_async_copy(meta_h.at[jbase + j], mb.at[b], msem[b]).wait()
        pltpu.make_async_copy(norm_h.at[jbase + j], nb.at[b], nsem[b]).wait()

    def start_gather(b):
        pltpu.async_copy(proj.at[gb.at[b]], rows.at[b], gsem[b])

    def wait_gather(b):
        pltpu.make_async_copy(proj.at[gb.at[b]], rows.at[b], gsem[b]).wait()

    def scale_rows(b):
        def scale(gg, c2):
            nv16 = nb[b, pl.ds(gg * 16, 16)]
            for k in range(16):
                nvk = jnp.full((16,), nv16[k], jnp.float32)
                row = gg * 16 + k
                for h in range(HALF // 16):
                    rows[b, row, pl.ds(h * 16, 16)] = (
                        rows[b, row, pl.ds(h * 16, 16)] * nvk)
            return c2
        lax.fori_loop(0, BATCH // 16, scale, 0)

    def scatter_rows(b):
        pltpu.sync_copy(rows.at[b], acc.at[mb.at[b, 2]], add=True)

    start_meta(0, 0)
    wait_meta(0, 0)
    build_idx(0, 0)
    start_gather(0)
    start_meta(1, 1)

    def pair(t, carry):
        for b in range(2):
            j = t * 2 + b
            b2 = 1 - b
            wait_gather(b)
            scale_rows(b)
            wait_meta(b2, j + 1)
            build_idx(b2, j + 1)
            start_gather(b2)
            scatter_rows(b)
            nxt = j + 2

            @pl.when(nxt < NBATCH)
            def _():
                start_meta(b, nxt)
        return carry
    lax.fori_loop(0, (NBATCH - 1) // 2, pair, 0)

    wait_gather(0)
    scale_rows(0)
    scatter_rows(0)

    plsc.subcore_barrier()

    obase = c * APAD + s * RPT
    off = 0
    for zc in ZCHUNKS:
        pltpu.sync_copy(acc.at[pl.ds(abase + off, zc)],
                        out.at[pl.ds(obase + off, zc)])
        off += zc


_sc_scatter_cache = []


def _get_sc_scatter():
    if not _sc_scatter_cache:
        _sc_scatter_cache.append(pl.kernel(
            _sc_body,
            out_type=jax.ShapeDtypeStruct((NC * APAD, HALF), jnp.float32),
            mesh=plsc.VectorSubcoreMesh(core_axis_name="c",
                                        subcore_axis_name="s",
                                        num_cores=NC, num_subcores=NS),
            scratch_types=[
                pltpu.VMEM_SHARED((APAD, HALF), jnp.float32),
                pltpu.VMEM((2, 3, BATCH), jnp.int32),
                pltpu.VMEM((2, BATCH), jnp.float32),
                pltpu.VMEM((2, BATCH), jnp.int32),
                pltpu.VMEM((2, BATCH, HALF), jnp.float32),
                pltpu.SemaphoreType.DMA,
                pltpu.SemaphoreType.DMA,
                pltpu.SemaphoreType.DMA,
                pltpu.SemaphoreType.DMA,
                pltpu.SemaphoreType.DMA,
                pltpu.SemaphoreType.DMA,
            ],
        ))
    return _sc_scatter_cache[0]


@jax.jit
def kernel(x, edge_index, edge_type, edge_norm, weight, w_comp):
    wfull_flat = pl.pallas_call(
        _wfull_body,
        out_shape=jax.ShapeDtypeStruct((NUM_RELS, IN_FEAT * OUT_FEAT),
                                       jnp.float32),
    )(w_comp, weight.reshape(NUM_BASES, IN_FEAT * OUT_FEAT))
    wfull = wfull_flat.reshape(NUM_RELS, IN_FEAT, OUT_FEAT)

    bn = 2000
    nb = N_NODES // bn
    proj = pl.pallas_call(
        _proj_body,
        grid=(nb, NUM_RELS, NC),
        in_specs=[
            pl.BlockSpec((bn, IN_FEAT), lambda i, r, h: (i, 0)),
            pl.BlockSpec((1, IN_FEAT, HALF), lambda i, r, h: (r, 0, h)),
        ],
        out_specs=pl.BlockSpec(
            (bn, HALF), lambda i, r, h: ((h * NUM_RELS + r) * nb + i, 0)),
        out_shape=jax.ShapeDtypeStruct((NC * NUM_RELS * N_NODES, HALF),
                                       jnp.float32),
    )(x, wfull)

    pad = EPAD - N_EDGES
    src = jnp.pad(edge_index[0], (0, pad)).astype(jnp.int32)
    dst = jnp.pad(edge_index[1], (0, pad)).astype(jnp.int32)
    etype = jnp.pad(edge_type, (0, pad)).astype(jnp.int32)
    norm = jnp.pad(edge_norm.reshape(N_EDGES).astype(jnp.float32), (0, pad))
    meta = jnp.stack([src.reshape(NS * NBATCH, BATCH),
                      etype.reshape(NS * NBATCH, BATCH),
                      dst.reshape(NS * NBATCH, BATCH)], axis=1)
    out2 = _get_sc_scatter()(proj, meta, norm.reshape(NS * NBATCH, BATCH))
    return (out2.reshape(NC, APAD, HALF)[:, :N_NODES]
            .transpose(1, 0, 2)
            .reshape(N_NODES, OUT_FEAT))


# speedup vs baseline: 2.7031x; 1.1015x over previous
import jax
import jax.numpy as jnp
from jax import lax
from jax.experimental import pallas as pl
from jax.experimental.pallas import tpu as pltpu
from jax.experimental.pallas import tpu_sc as plsc

N_NODES = 10000
N_EDGES = 160000
IN_FEAT = 256
OUT_FEAT = 256
NUM_RELS = 16
NUM_BASES = 8

HALF = 128
NC = 2
NS = 16
BATCH = 128
NBATCH = 79
EPT = NBATCH * BATCH
EPAD = EPT * NS
APAD = 10112
RPT = APAD // NS
ZCHUNKS = (128, 128, 128, 128, 120)


def _wfull_body(c_ref, w_ref, o_ref):
    o_ref[...] = jnp.dot(c_ref[...], w_ref[...],
                         preferred_element_type=jnp.float32)


def _proj_body(x_ref, wf_ref, o_ref):
    o_ref[...] = jnp.dot(x_ref[...], wf_ref[0],
                         preferred_element_type=jnp.float32)


def _sc_body(proj, meta_h, norm_h, out,
             acc, mb, nb, gb, db, rows,
             msem0, msem1, nsem0, nsem1, gsem0, gsem1, ssem0, ssem1):
    c = lax.axis_index("c")
    s = lax.axis_index("s")
    zf = jnp.zeros((16,), jnp.float32)
    msem = (msem0, msem1)
    nsem = (nsem0, nsem1)
    gsem = (gsem0, gsem1)
    ssem = (ssem0, ssem1)

    def zrow(jj, carry):
        for k in range(HALF // 16):
            rows[0, jj, pl.ds(k * 16, 16)] = zf
        return carry
    lax.fori_loop(0, BATCH, zrow, 0)

    abase = s * RPT
    off = 0
    for zc in ZCHUNKS:
        pltpu.sync_copy(rows.at[0, pl.ds(0, zc)],
                        acc.at[pl.ds(abase + off, zc)])
        off += zc

    plsc.subcore_barrier()

    crn = c * (NUM_RELS * N_NODES)
    jbase = s * NBATCH

    def build_idx(b, j):
        del j
        for t in range(BATCH // 16):
            sl = pl.ds(t * 16, 16)
            gb[b, sl] = mb[b, 1, sl] * N_NODES + mb[b, 0, sl] + crn

    def start_meta(b, j):
        pltpu.async_copy(meta_h.at[jbase + j], mb.at[b], msem[b])
        pltpu.async_copy(norm_h.at[jbase + j], nb.at[b], nsem[b])

    def wait_meta(b, j):
        pltpu.make_async_copy(meta_h.at[jbase + j], mb.at[b], msem[b]).wait()
        pltpu.make_async_copy(norm_h.at[jbase + j], nb.at[b], nsem[b]).wait()

    def start_gather(b):
        pltpu.async_copy(proj.at[gb.at[b]], rows.at[b], gsem[b])

    def wait_gather(b):
        pltpu.make_async_copy(proj.at[gb.at[b]], rows.at[b], gsem[b]).wait()

    def scale_rows(b):
        def scale(gg, c2):
            nv16 = nb[b, pl.ds(gg * 16, 16)]
            for k in range(16):
                nvk = jnp.full((16,), nv16[k], jnp.float32)
                row = gg * 16 + k
                for h in range(HALF // 16):
                    rows[b, row, pl.ds(h * 16, 16)] = (
                        rows[b, row, pl.ds(h * 16, 16)] * nvk)
            return c2
        lax.fori_loop(0, BATCH // 16, scale, 0)

    def copy_dst(b):
        for t in range(BATCH // 16):
            sl = pl.ds(t * 16, 16)
            db[b, sl] = mb[b, 2, sl]

    def start_scatter(b):
        pltpu.async_copy(rows.at[b], acc.at[db.at[b]], ssem[b], add=True)

    def wait_scatter(b):
        pltpu.make_async_copy(rows.at[b], acc.at[db.at[b]], ssem[b]).wait()

    start_meta(0, 0)
    wait_meta(0, 0)
    build_idx(0, 0)
    start_gather(0)
    start_meta(1, 1)

    def pair(t, carry):
        for b in range(2):
            j = t * 2 + b
            b2 = 1 - b
            wait_gather(b)
            wait_meta(b2, j + 1)
            build_idx(b2, j + 1)
            if b == 0:
                @pl.when(t > 0)
                def _():
                    wait_scatter(b2)
            else:
                wait_scatter(b2)
            start_gather(b2)
            scale_rows(b)
            copy_dst(b)
            start_scatter(b)
            nxt = j + 2

            @pl.when(nxt < NBATCH)
            def _():
                start_meta(b, nxt)
        return carry
    lax.fori_loop(0, (NBATCH - 1) // 2, pair, 0)

    wait_gather(0)
    scale_rows(0)
    copy_dst(0)
    start_scatter(0)
    wait_scatter(0)
    wait_scatter(1)

    plsc.subcore_barrier()

    obase = c * APAD + s * RPT
    off = 0
    for zc in ZCHUNKS:
        pltpu.sync_copy(acc.at[pl.ds(abase + off, zc)],
                        out.at[pl.ds(obase + off, zc)])
        off += zc


_sc_scatter_cache = []


def _get_sc_scatter():
    if not _sc_scatter_cache:
        _sc_scatter_cache.append(pl.kernel(
            _sc_body,
            out_type=jax.ShapeDtypeStruct((NC * APAD, HALF), jnp.float32),
            mesh=plsc.VectorSubcoreMesh(core_axis_name="c",
                                        subcore_axis_name="s",
                                        num_cores=NC, num_subcores=NS),
            scratch_types=[
                pltpu.VMEM_SHARED((APAD, HALF), jnp.float32),
                pltpu.VMEM((2, 3, BATCH), jnp.int32),
                pltpu.VMEM((2, BATCH), jnp.float32),
                pltpu.VMEM((2, BATCH), jnp.int32),
                pltpu.VMEM((2, BATCH), jnp.int32),
                pltpu.VMEM((2, BATCH, HALF), jnp.float32),
                pltpu.SemaphoreType.DMA,
                pltpu.SemaphoreType.DMA,
                pltpu.SemaphoreType.DMA,
                pltpu.SemaphoreType.DMA,
                pltpu.SemaphoreType.DMA,
                pltpu.SemaphoreType.DMA,
                pltpu.SemaphoreType.DMA,
                pltpu.SemaphoreType.DMA,
            ],
        ))
    return _sc_scatter_cache[0]


@jax.jit
def kernel(x, edge_index, edge_type, edge_norm, weight, w_comp):
    wfull_flat = pl.pallas_call(
        _wfull_body,
        out_shape=jax.ShapeDtypeStruct((NUM_RELS, IN_FEAT * OUT_FEAT),
                                       jnp.float32),
    )(w_comp, weight.reshape(NUM_BASES, IN_FEAT * OUT_FEAT))
    wfull = wfull_flat.reshape(NUM_RELS, IN_FEAT, OUT_FEAT)

    bn = 2000
    nb = N_NODES // bn
    proj = pl.pallas_call(
        _proj_body,
        grid=(nb, NUM_RELS, NC),
        in_specs=[
            pl.BlockSpec((bn, IN_FEAT), lambda i, r, h: (i, 0)),
            pl.BlockSpec((1, IN_FEAT, HALF), lambda i, r, h: (r, 0, h)),
        ],
        out_specs=pl.BlockSpec(
            (bn, HALF), lambda i, r, h: ((h * NUM_RELS + r) * nb + i, 0)),
        out_shape=jax.ShapeDtypeStruct((NC * NUM_RELS * N_NODES, HALF),
                                       jnp.float32),
    )(x, wfull)

    pad = EPAD - N_EDGES
    src = jnp.pad(edge_index[0], (0, pad)).astype(jnp.int32)
    dst = jnp.pad(edge_index[1], (0, pad)).astype(jnp.int32)
    etype = jnp.pad(edge_type, (0, pad)).astype(jnp.int32)
    norm = jnp.pad(edge_norm.reshape(N_EDGES).astype(jnp.float32), (0, pad))
    meta = jnp.stack([src.reshape(NS * NBATCH, BATCH),
                      etype.reshape(NS * NBATCH, BATCH),
                      dst.reshape(NS * NBATCH, BATCH)], axis=1)
    out2 = _get_sc_scatter()(proj, meta, norm.reshape(NS * NBATCH, BATCH))
    return (out2.reshape(NC, APAD, HALF)[:, :N_NODES]
            .transpose(1, 0, 2)
            .reshape(N_NODES, OUT_FEAT))


# speedup vs baseline: 2.7531x; 1.0185x over previous
import jax
import jax.numpy as jnp
from jax import lax
from jax.experimental import pallas as pl
from jax.experimental.pallas import tpu as pltpu
from jax.experimental.pallas import tpu_sc as plsc

N_NODES = 10000
N_EDGES = 160000
IN_FEAT = 256
OUT_FEAT = 256
NUM_RELS = 16
NUM_BASES = 8

HALF = 128
NC = 2
NS = 16
BATCH = 128
NBATCH = 79
EPT = NBATCH * BATCH
EPAD = EPT * NS
APAD = 10112
RPT = APAD // NS
ZCHUNKS = (128, 128, 128, 128, 120)


def _wfull_body(c_ref, w_ref, o_ref):
    o_ref[...] = jnp.dot(c_ref[...], w_ref[...],
                         preferred_element_type=jnp.float32)


def _proj_body(x_ref, wf_ref, o_ref):
    o_ref[...] = jnp.dot(x_ref[...], wf_ref[0],
                         preferred_element_type=jnp.float32)


def _sc_body(proj, meta_h, norm_h, out,
             acc, mb, nb, gb, db, rows,
             msem0, msem1, nsem0, nsem1, gsem0, gsem1, ssem0, ssem1):
    c = lax.axis_index("c")
    s = lax.axis_index("s")
    zf = jnp.zeros((16,), jnp.float32)
    msem = (msem0, msem1)
    nsem = (nsem0, nsem1)
    gsem = (gsem0, gsem1)
    ssem = (ssem0, ssem1)

    def zrow(jj, carry):
        for k in range(HALF // 16):
            rows[0, jj, pl.ds(k * 16, 16)] = zf
        return carry
    lax.fori_loop(0, BATCH, zrow, 0)

    abase = s * RPT
    off = 0
    for zc in ZCHUNKS:
        pltpu.sync_copy(rows.at[0, pl.ds(0, zc)],
                        acc.at[pl.ds(abase + off, zc)])
        off += zc

    plsc.subcore_barrier()

    crn = c * (NUM_RELS * N_NODES)
    jbase = s * NBATCH

    def build_idx(b, j):
        del j
        for t in range(BATCH // 16):
            sl = pl.ds(t * 16, 16)
            gb[b, sl] = mb[b, 1, sl] * N_NODES + mb[b, 0, sl] + crn

    def start_meta(b, j):
        pltpu.async_copy(meta_h.at[jbase + j], mb.at[b], msem[b])
        pltpu.async_copy(norm_h.at[jbase + j], nb.at[b], nsem[b])

    def wait_meta(b, j):
        pltpu.make_async_copy(meta_h.at[jbase + j], mb.at[b], msem[b]).wait()
        pltpu.make_async_copy(norm_h.at[jbase + j], nb.at[b], nsem[b]).wait()

    def start_gather(b):
        pltpu.async_copy(proj.at[gb.at[b]], rows.at[b], gsem[b])

    def wait_gather(b):
        pltpu.make_async_copy(proj.at[gb.at[b]], rows.at[b], gsem[b]).wait()

    def scale_rows(b):
        def scale(gg, c2):
            nv16 = nb[b, pl.ds(gg * 16, 16)]
            for k in range(16):
                nvk = jnp.full((16,), nv16[k], jnp.float32)
                row = gg * 16 + k
                for h in range(HALF // 16):
                    rows[b, row, pl.ds(h * 16, 16)] = (
                        rows[b, row, pl.ds(h * 16, 16)] * nvk)
            return c2
        lax.fori_loop(0, BATCH // 16, scale, 0)

    def copy_dst(b):
        for t in range(BATCH // 16):
            sl = pl.ds(t * 16, 16)
            db[b, sl] = mb[b, 2, sl]

    def start_scatter(b):
        pltpu.async_copy(rows.at[b], acc.at[db.at[b]], ssem[b], add=True)

    def wait_scatter(b):
        pltpu.make_async_copy(rows.at[b], acc.at[db.at[b]], ssem[b]).wait()

    start_meta(0, 0)
    wait_meta(0, 0)
    build_idx(0, 0)
    start_gather(0)
    start_meta(1, 1)

    def pair(t, carry):
        for b in range(2):
            j = t * 2 + b
            b2 = 1 - b
            wait_gather(b)
            wait_meta(b2, j + 1)
            build_idx(b2, j + 1)
            if b == 0:
                @pl.when(t > 0)
                def _():
                    wait_scatter(b2)
            else:
                wait_scatter(b2)
            start_gather(b2)
            scale_rows(b)
            copy_dst(b)
            start_scatter(b)
            nxt = j + 2

            @pl.when(nxt < NBATCH)
            def _():
                start_meta(b, nxt)
        return carry
    lax.fori_loop(0, (NBATCH - 1) // 2, pair, 0)

    wait_gather(0)
    scale_rows(0)
    copy_dst(0)
    start_scatter(0)
    wait_scatter(0)
    wait_scatter(1)

    plsc.subcore_barrier()

    obase = c * APAD + s * RPT
    off = 0
    for zc in ZCHUNKS:
        pltpu.sync_copy(acc.at[pl.ds(abase + off, zc)],
                        out.at[pl.ds(obase + off, zc)])
        off += zc


_sc_scatter_cache = []


def _get_sc_scatter():
    if not _sc_scatter_cache:
        _sc_scatter_cache.append(pl.kernel(
            _sc_body,
            out_type=jax.ShapeDtypeStruct((NC * APAD, HALF), jnp.float32),
            mesh=plsc.VectorSubcoreMesh(core_axis_name="c",
                                        subcore_axis_name="s",
                                        num_cores=NC, num_subcores=NS),
            scratch_types=[
                pltpu.VMEM_SHARED((APAD, HALF), jnp.float32),
                pltpu.VMEM((2, 3, BATCH), jnp.int32),
                pltpu.VMEM((2, BATCH), jnp.float32),
                pltpu.VMEM((2, BATCH), jnp.int32),
                pltpu.VMEM((2, BATCH), jnp.int32),
                pltpu.VMEM((2, BATCH, HALF), jnp.float32),
                pltpu.SemaphoreType.DMA,
                pltpu.SemaphoreType.DMA,
                pltpu.SemaphoreType.DMA,
                pltpu.SemaphoreType.DMA,
                pltpu.SemaphoreType.DMA,
                pltpu.SemaphoreType.DMA,
                pltpu.SemaphoreType.DMA,
                pltpu.SemaphoreType.DMA,
            ],
        ))
    return _sc_scatter_cache[0]


@jax.jit
def kernel(x, edge_index, edge_type, edge_norm, weight, w_comp):
    wfull_flat = pl.pallas_call(
        _wfull_body,
        out_shape=jax.ShapeDtypeStruct((NUM_RELS, IN_FEAT * OUT_FEAT),
                                       jnp.float32),
    )(w_comp, weight.reshape(NUM_BASES, IN_FEAT * OUT_FEAT))
    wfull = wfull_flat.reshape(NUM_RELS, IN_FEAT, OUT_FEAT)

    bn = 2000
    nb = N_NODES // bn
    xb = x.astype(jnp.bfloat16)
    wb = wfull.astype(jnp.bfloat16)
    proj = pl.pallas_call(
        _proj_body,
        grid=(nb, NUM_RELS, NC),
        in_specs=[
            pl.BlockSpec((bn, IN_FEAT), lambda i, r, h: (i, 0)),
            pl.BlockSpec((1, IN_FEAT, HALF), lambda i, r, h: (r, 0, h)),
        ],
        out_specs=pl.BlockSpec(
            (bn, HALF), lambda i, r, h: ((h * NUM_RELS + r) * nb + i, 0)),
        out_shape=jax.ShapeDtypeStruct((NC * NUM_RELS * N_NODES, HALF),
                                       jnp.float32),
    )(xb, wb)

    pad = EPAD - N_EDGES
    src = jnp.pad(edge_index[0], (0, pad)).astype(jnp.int32)
    dst = jnp.pad(edge_index[1], (0, pad)).astype(jnp.int32)
    etype = jnp.pad(edge_type, (0, pad)).astype(jnp.int32)
    norm = jnp.pad(edge_norm.reshape(N_EDGES).astype(jnp.float32), (0, pad))
    meta = jnp.stack([src.reshape(NS * NBATCH, BATCH),
                      etype.reshape(NS * NBATCH, BATCH),
                      dst.reshape(NS * NBATCH, BATCH)], axis=1)
    out2 = _get_sc_scatter()(proj, meta, norm.reshape(NS * NBATCH, BATCH))
    return (out2.reshape(NC, APAD, HALF)[:, :N_NODES]
            .transpose(1, 0, 2)
            .reshape(N_NODES, OUT_FEAT))


# speedup vs baseline: 3.1946x; 1.1604x over previous
import jax
import jax.numpy as jnp
from jax import lax
from jax.experimental import pallas as pl
from jax.experimental.pallas import tpu as pltpu
from jax.experimental.pallas import tpu_sc as plsc

N_NODES = 10000
N_EDGES = 160000
IN_FEAT = 256
OUT_FEAT = 256
NUM_RELS = 16
NUM_BASES = 8

HALF = 128
NC = 2
NS = 16
BATCH = 128
NBATCH = 79
EPT = NBATCH * BATCH
EPAD = EPT * NS
APAD = 10112
RPT = APAD // NS
ZCHUNKS = (128, 128, 128, 128, 120)


def _wfull_body(c_ref, w_ref, o_ref):
    o_ref[...] = jnp.dot(c_ref[...], w_ref[...],
                         preferred_element_type=jnp.float32)


def _proj_body(x_ref, wf_ref, o_ref):
    o_ref[...] = jnp.dot(x_ref[...], wf_ref[0],
                         preferred_element_type=jnp.float32)


def _sc_body(proj, meta_h, norm_h, out,
             acc, mb, nball, gb, db, rows,
             msem0, msem1, gsem0, gsem1, ssem0, ssem1):
    c = lax.axis_index("c")
    s = lax.axis_index("s")
    zf = jnp.zeros((16,), jnp.float32)
    msem = (msem0, msem1)
    gsem = (gsem0, gsem1)
    ssem = (ssem0, ssem1)

    def zrow(jj, carry):
        for k in range(HALF // 16):
            rows[0, jj, pl.ds(k * 16, 16)] = zf
        return carry
    lax.fori_loop(0, BATCH, zrow, 0)

    abase = s * RPT
    off = 0
    for zc in ZCHUNKS:
        pltpu.sync_copy(rows.at[0, pl.ds(0, zc)],
                        acc.at[pl.ds(abase + off, zc)])
        off += zc

    plsc.subcore_barrier()

    crn = c * (NUM_RELS * N_NODES)
    jbase = s * NBATCH

    pltpu.sync_copy(norm_h.at[s], nball)

    def build_idx(b, j):
        del j
        for t in range(BATCH // 16):
            sl = pl.ds(t * 16, 16)
            gb[b, sl] = mb[b, 1, sl] * N_NODES + mb[b, 0, sl] + crn

    def start_meta(b, j):
        pltpu.async_copy(meta_h.at[jbase + j], mb.at[b], msem[b])

    def wait_meta(b, j):
        pltpu.make_async_copy(meta_h.at[jbase + j], mb.at[b], msem[b]).wait()

    def start_gather(b):
        pltpu.async_copy(proj.at[gb.at[b]], rows.at[b], gsem[b])

    def wait_gather(b):
        pltpu.make_async_copy(proj.at[gb.at[b]], rows.at[b], gsem[b]).wait()

    def scale_rows(b, j):
        def scale(gg, c2):
            nv16 = nball[j, pl.ds(gg * 16, 16)]
            for k in range(16):
                nvk = jnp.full((16,), nv16[k], jnp.float32)
                row = gg * 16 + k
                for h in range(HALF // 16):
                    rows[b, row, pl.ds(h * 16, 16)] = (
                        rows[b, row, pl.ds(h * 16, 16)] * nvk)
            return c2
        lax.fori_loop(0, BATCH // 16, scale, 0)

    def copy_dst(b):
        for t in range(BATCH // 16):
            sl = pl.ds(t * 16, 16)
            db[b, sl] = mb[b, 2, sl]

    def start_scatter(b):
        pltpu.async_copy(rows.at[b], acc.at[db.at[b]], ssem[b], add=True)

    def wait_scatter(b):
        pltpu.make_async_copy(rows.at[b], acc.at[db.at[b]], ssem[b]).wait()

    start_meta(0, 0)
    wait_meta(0, 0)
    build_idx(0, 0)
    start_gather(0)
    start_meta(1, 1)

    def pair(t, carry):
        for b in range(2):
            j = t * 2 + b
            b2 = 1 - b
            wait_gather(b)
            wait_meta(b2, j + 1)
            build_idx(b2, j + 1)
            if b == 0:
                @pl.when(t > 0)
                def _():
                    wait_scatter(b2)
            else:
                wait_scatter(b2)
            start_gather(b2)
            scale_rows(b, j)
            copy_dst(b)
            start_scatter(b)
            nxt = j + 2

            @pl.when(nxt < NBATCH)
            def _():
                start_meta(b, nxt)
        return carry
    lax.fori_loop(0, (NBATCH - 1) // 2, pair, 0)

    wait_gather(0)
    scale_rows(0, NBATCH - 1)
    copy_dst(0)
    start_scatter(0)
    wait_scatter(0)
    wait_scatter(1)

    plsc.subcore_barrier()

    obase = c * APAD + s * RPT
    off = 0
    for zc in ZCHUNKS:
        pltpu.sync_copy(acc.at[pl.ds(abase + off, zc)],
                        out.at[pl.ds(obase + off, zc)])
        off += zc


_sc_scatter_cache = []


def _get_sc_scatter():
    if not _sc_scatter_cache:
        _sc_scatter_cache.append(pl.kernel(
            _sc_body,
            out_type=jax.ShapeDtypeStruct((NC * APAD, HALF), jnp.float32),
            mesh=plsc.VectorSubcoreMesh(core_axis_name="c",
                                        subcore_axis_name="s",
                                        num_cores=NC, num_subcores=NS),
            scratch_types=[
                pltpu.VMEM_SHARED((APAD, HALF), jnp.float32),
                pltpu.VMEM((2, 3, BATCH), jnp.int32),
                pltpu.VMEM((NBATCH, BATCH), jnp.float32),
                pltpu.VMEM((2, BATCH), jnp.int32),
                pltpu.VMEM((2, BATCH), jnp.int32),
                pltpu.VMEM((2, BATCH, HALF), jnp.float32),
                pltpu.SemaphoreType.DMA,
                pltpu.SemaphoreType.DMA,
                pltpu.SemaphoreType.DMA,
                pltpu.SemaphoreType.DMA,
                pltpu.SemaphoreType.DMA,
                pltpu.SemaphoreType.DMA,
            ],
        ))
    return _sc_scatter_cache[0]


@jax.jit
def kernel(x, edge_index, edge_type, edge_norm, weight, w_comp):
    wfull_flat = pl.pallas_call(
        _wfull_body,
        out_shape=jax.ShapeDtypeStruct((NUM_RELS, IN_FEAT * OUT_FEAT),
                                       jnp.float32),
    )(w_comp, weight.reshape(NUM_BASES, IN_FEAT * OUT_FEAT))
    wfull = wfull_flat.reshape(NUM_RELS, IN_FEAT, OUT_FEAT)

    bn = 5000
    nb = N_NODES // bn
    xb = x.astype(jnp.bfloat16)
    wb = wfull.astype(jnp.bfloat16)
    proj = pl.pallas_call(
        _proj_body,
        grid=(nb, NUM_RELS, NC),
        in_specs=[
            pl.BlockSpec((bn, IN_FEAT), lambda i, r, h: (i, 0)),
            pl.BlockSpec((1, IN_FEAT, HALF), lambda i, r, h: (r, 0, h)),
        ],
        out_specs=pl.BlockSpec(
            (bn, HALF), lambda i, r, h: ((h * NUM_RELS + r) * nb + i, 0)),
        out_shape=jax.ShapeDtypeStruct((NC * NUM_RELS * N_NODES, HALF),
                                       jnp.float32),
    )(xb, wb)

    pad = EPAD - N_EDGES
    src = jnp.pad(edge_index[0], (0, pad)).astype(jnp.int32)
    dst = jnp.pad(edge_index[1], (0, pad)).astype(jnp.int32)
    etype = jnp.pad(edge_type, (0, pad)).astype(jnp.int32)
    norm = jnp.pad(edge_norm.reshape(N_EDGES).astype(jnp.float32), (0, pad))
    meta = jnp.stack([src.reshape(NS * NBATCH, BATCH),
                      etype.reshape(NS * NBATCH, BATCH),
                      dst.reshape(NS * NBATCH, BATCH)], axis=1)
    out2 = _get_sc_scatter()(proj, meta, norm.reshape(NS, NBATCH, BATCH))
    return (out2.reshape(NC, APAD, HALF)[:, :N_NODES]
            .transpose(1, 0, 2)
            .reshape(N_NODES, OUT_FEAT))
